# baseline (speedup 1.0000x reference)
import functools

import jax
import jax.numpy as jnp
from jax import lax
from jax.experimental import pallas as pl
from jax.experimental.pallas import tpu as pltpu
from jax.experimental.pallas import tpu_sc as plsc

EMB = 64
SEQ = 200
NC, NS, L = 2, 16, 16
NW = NC * NS
CHUNK = 128
VPR = EMB // L


def _emb_kernel(n_rows):
    rows_per_w = n_rows // NW
    n_chunks = rows_per_w // CHUNK
    mesh = plsc.VectorSubcoreMesh(
        core_axis_name="c", subcore_axis_name="s", num_cores=NC, num_subcores=NS
    )

    @functools.partial(
        pl.kernel,
        out_type=jax.ShapeDtypeStruct((n_rows, EMB), jnp.float32),
        mesh=mesh,
        scratch_types=[
            pltpu.VMEM((CHUNK,), jnp.int32),
            pltpu.VMEM((CHUNK, EMB), jnp.float32),
            pltpu.VMEM((2 * SEQ, EMB), jnp.float32),
            pltpu.SemaphoreType.DMA,
        ],
        compiler_params=pltpu.CompilerParams(use_tc_tiling_on_sc=False),
    )
    def body(tok_hbm, tab_hbm, pos_hbm, out_hbm, idx_v, rows_v, pos2_v, sem):
        wid = lax.axis_index("s") * NC + lax.axis_index("c")
        base_w = wid * rows_per_w
        pltpu.sync_copy(pos_hbm, pos2_v.at[pl.ds(0, SEQ)])
        pltpu.sync_copy(pos_hbm, pos2_v.at[pl.ds(SEQ, SEQ)])

        def chunk_body(i, carry):
            base = base_w + i * CHUNK
            pltpu.sync_copy(tok_hbm.at[pl.ds(base, CHUNK)], idx_v)
            pltpu.async_copy(tab_hbm.at[idx_v], rows_v, sem).wait()
            off = (i * CHUNK) % SEQ

            def row_body(r, c2):
                for j in range(VPR):
                    sl = pl.ds(j * L, L)
                    plsc.addupdate(rows_v.at[r, sl], pos2_v[off + r, sl])
                return c2

            lax.fori_loop(0, CHUNK, row_body, 0)
            pltpu.sync_copy(rows_v, out_hbm.at[pl.ds(base, CHUNK)])
            return carry

        lax.fori_loop(0, n_chunks, chunk_body, 0)

    return body


def kernel(tokens, static_table, pos_table):
    b, s = tokens.shape
    toks = tokens.reshape(-1).astype(jnp.int32)
    out = _emb_kernel(b * s)(toks, static_table, pos_table)
    return out.reshape(b, s, EMB)


# speedup vs baseline: 1.1571x; 1.1571x over previous
import functools

import jax
import jax.numpy as jnp
from jax import lax
from jax.experimental import pallas as pl
from jax.experimental.pallas import tpu as pltpu
from jax.experimental.pallas import tpu_sc as plsc

EMB = 64
SEQ = 200
NC, NS, L = 2, 16, 16
NW = NC * NS
CHUNK = 128
VPR = EMB // L
NBUF = 4


def _emb_kernel(n_rows):
    rows_per_w = n_rows // NW
    n_chunks = rows_per_w // CHUNK
    n_groups = n_chunks // NBUF
    mesh = plsc.VectorSubcoreMesh(
        core_axis_name="c", subcore_axis_name="s", num_cores=NC, num_subcores=NS
    )

    @functools.partial(
        pl.kernel,
        out_type=jax.ShapeDtypeStruct((n_rows, EMB), jnp.float32),
        mesh=mesh,
        scratch_types=[
            pltpu.VMEM((n_chunks, CHUNK), jnp.int32),
            pltpu.VMEM((NBUF, CHUNK, EMB), jnp.float32),
            pltpu.VMEM((NBUF, CHUNK, EMB), jnp.float32),
            pltpu.VMEM((SEQ + CHUNK, EMB), jnp.float32),
            pltpu.SemaphoreType.DMA((NBUF,)),
            pltpu.SemaphoreType.DMA((NBUF,)),
        ],
        compiler_params=pltpu.CompilerParams(use_tc_tiling_on_sc=False),
    )
    def body(tok_hbm, tab_hbm, pos_hbm, out_hbm, idx_all, gbuf, wbuf, pos2_v,
             gsem, wsem):
        wid = lax.axis_index("s") * NC + lax.axis_index("c")
        base_w = wid * rows_per_w
        pltpu.sync_copy(tok_hbm.at[pl.ds(wid * n_chunks, n_chunks)], idx_all)
        pltpu.sync_copy(pos_hbm, pos2_v.at[pl.ds(0, SEQ)])
        pltpu.sync_copy(pos_hbm.at[pl.ds(0, CHUNK)],
                        pos2_v.at[pl.ds(SEQ, CHUNK)])

        def gather(i, b):
            pltpu.async_copy(
                tab_hbm.at[idx_all.at[i]], gbuf.at[b], gsem.at[b])

        def gather_wait(i, b):
            pltpu.make_async_copy(
                tab_hbm.at[idx_all.at[i]], gbuf.at[b], gsem.at[b]).wait()

        def write(i, b):
            pltpu.async_copy(
                wbuf.at[b], out_hbm.at[pl.ds(base_w + i * CHUNK, CHUNK)],
                wsem.at[b])

        def write_wait(i, b):
            pltpu.make_async_copy(
                wbuf.at[b], out_hbm.at[pl.ds(base_w + i * CHUNK, CHUNK)],
                wsem.at[b]).wait()

        for b in range(NBUF):
            gather(b, b)

        def group_body(q, carry):
            for b in range(NBUF):
                i = q * NBUF + b
                gather_wait(i, b)
                off = (i * CHUNK) % SEQ

                @pl.when(i >= NBUF)
                def _():
                    write_wait(i - NBUF, b)

                def addrow(r, c2):
                    for j in range(VPR):
                        sl = pl.ds(j * L, L)
                        wbuf[b, r, sl] = gbuf[b, r, sl] + pos2_v[off + r, sl]
                    return c2

                lax.fori_loop(0, CHUNK, addrow, 0)

                write(i, b)

                @pl.when(i + NBUF < n_chunks)
                def _():
                    gather(i + NBUF, b)
            return carry

        lax.fori_loop(0, n_groups, group_body, 0)
        for b in range(NBUF):
            write_wait(n_chunks - NBUF + b, b)

    return body


def kernel(tokens, static_table, pos_table):
    b, s = tokens.shape
    toks = tokens.reshape(-1, CHUNK).astype(jnp.int32)
    out = _emb_kernel(b * s)(toks, static_table, pos_table)
    return out.reshape(b, s, EMB)


# speedup vs baseline: 1.1682x; 1.0096x over previous
import functools

import jax
import jax.numpy as jnp
from jax import lax
from jax.experimental import pallas as pl
from jax.experimental.pallas import tpu as pltpu
from jax.experimental.pallas import tpu_sc as plsc

EMB = 64
SEQ = 200
NC, NS, L = 2, 16, 16
NW = NC * NS
CHUNK = 128
VPR = EMB // L
NBUF = 4


def _emb_kernel(n_rows):
    rows_per_w = n_rows // NW
    n_chunks = rows_per_w // CHUNK
    n_groups = n_chunks // NBUF
    mesh = plsc.VectorSubcoreMesh(
        core_axis_name="c", subcore_axis_name="s", num_cores=NC, num_subcores=NS
    )

    @functools.partial(
        pl.kernel,
        out_type=jax.ShapeDtypeStruct((n_rows, EMB), jnp.float32),
        mesh=mesh,
        scratch_types=[
            pltpu.VMEM((n_chunks, CHUNK), jnp.int32),
            pltpu.VMEM((NBUF, CHUNK, EMB), jnp.float32),
            pltpu.VMEM((NBUF, CHUNK, EMB), jnp.float32),
            pltpu.VMEM((SEQ + CHUNK, EMB), jnp.float32),
            pltpu.SemaphoreType.DMA((NBUF,)),
            pltpu.SemaphoreType.DMA((NBUF,)),
        ],
        compiler_params=pltpu.CompilerParams(use_tc_tiling_on_sc=False),
    )
    def body(tok_hbm, tab_hbm, pos_hbm, out_hbm, idx_all, gbuf, wbuf, pos2_v,
             gsem, wsem):
        wid = lax.axis_index("s") * NC + lax.axis_index("c")
        base_w = wid * rows_per_w
        pltpu.sync_copy(tok_hbm.at[pl.ds(wid * n_chunks, n_chunks)], idx_all)
        pltpu.sync_copy(pos_hbm, pos2_v.at[pl.ds(0, SEQ)])
        pltpu.sync_copy(pos_hbm.at[pl.ds(0, CHUNK)],
                        pos2_v.at[pl.ds(SEQ, CHUNK)])

        def gather(i, b):
            pltpu.async_copy(
                tab_hbm.at[idx_all.at[i]], gbuf.at[b], gsem.at[b])

        def gather_wait(i, b):
            pltpu.make_async_copy(
                tab_hbm.at[idx_all.at[i]], gbuf.at[b], gsem.at[b]).wait()

        def write(i, b):
            pltpu.async_copy(
                wbuf.at[b], out_hbm.at[pl.ds(base_w + i * CHUNK, CHUNK)],
                wsem.at[b])

        def write_wait(i, b):
            pltpu.make_async_copy(
                wbuf.at[b], out_hbm.at[pl.ds(base_w + i * CHUNK, CHUNK)],
                wsem.at[b]).wait()

        for b in range(NBUF):
            gather(b, b)

        def group_body(q, carry):
            for b in range(NBUF):
                i = q * NBUF + b
                gather_wait(i, b)
                off = (i * CHUNK) % SEQ

                @pl.when(i >= NBUF)
                def _():
                    write_wait(i - NBUF, b)

                @pl.loop(0, CHUNK, unroll=8)
                def addrow(r):
                    for j in range(VPR):
                        sl = pl.ds(j * L, L)
                        wbuf[b, r, sl] = gbuf[b, r, sl] + pos2_v[off + r, sl]

                write(i, b)

                @pl.when(i + NBUF < n_chunks)
                def _():
                    gather(i + NBUF, b)
            return carry

        lax.fori_loop(0, n_groups, group_body, 0)
        for b in range(NBUF):
            write_wait(n_chunks - NBUF + b, b)

    return body


def kernel(tokens, static_table, pos_table):
    b, s = tokens.shape
    toks = tokens.reshape(-1, CHUNK).astype(jnp.int32)
    out = _emb_kernel(b * s)(toks, static_table, pos_table)
    return out.reshape(b, s, EMB)
